# baseline (speedup 1.0000x reference)
import jax
import jax.numpy as jnp
from jax import lax
from jax.experimental import pallas as pl

_NUM_TOKENS = 20
_TEMP = 0.1


def _tables_body(seq_ref, w1_ref, w2_ref, gt_ref, rs_ref, r20_ref, sm_ref):
    L = seq_ref.shape[0]
    V = _NUM_TOKENS
    W1 = w1_ref[...]
    w2 = w2_ref[...]
    inv_l = 1.0 / L
    w1b16 = W1.astype(jnp.bfloat16)
    W1b = w1b16.astype(jnp.float32)
    r20 = jnp.maximum(W1b, 0.0)

    M = jnp.where(W1b > 0.0, w2 * inv_l, 0.0)
    Mb = M.astype(jnp.bfloat16).astype(jnp.float32)
    DT = jnp.concatenate(
        [jnp.sum(W1b * Mb[t:t + 1, :], axis=1, keepdims=True)
         for t in range(V)], axis=1)
    eye = (lax.broadcasted_iota(jnp.int32, (V, V), 0)
           == lax.broadcasted_iota(jnp.int32, (V, V), 1))
    ddiag_row = jnp.sum(jnp.where(eye, DT, 0.0), axis=0, keepdims=True)
    e_row = jnp.sum(jnp.exp((DT - ddiag_row) / _TEMP), axis=0, keepdims=True)

    seq = seq_ref[...]
    ohf = (lax.broadcasted_iota(jnp.int32, (L, V), 1) == seq).astype(jnp.float32)
    rows = lax.dot_general(ohf, DT, (((1,), (1,)), ((), ())),
                           preferred_element_type=jnp.float32,
                           precision=lax.Precision.HIGHEST)
    rseq = lax.dot_general(ohf, r20, (((1,), (0,)), ((), ())),
                           preferred_element_type=jnp.float32,
                           precision=lax.Precision.HIGHEST)
    dll = jnp.sum(rows * ohf, axis=1, keepdims=True)
    logits2d = (rows - dll) / _TEMP
    e_seq = jnp.sum(ohf * e_row, axis=1, keepdims=True)
    z_src = jnp.sum(e_seq, axis=0, keepdims=True)

    pooled_src = jnp.sum(rseq, axis=0, keepdims=True) * inv_l
    w2b = w2.astype(jnp.bfloat16).astype(jnp.float32)
    psb = pooled_src.astype(jnp.bfloat16).astype(jnp.float32)
    s_src = jnp.sum(psb * w2b, axis=1, keepdims=True)

    gt_ref[...] = jnp.concatenate(
        [logits2d, e_seq, jnp.zeros((L, 11), jnp.float32)], axis=1)
    rs_ref[...] = rseq
    r20_ref[...] = r20
    sm_ref[...] = jnp.concatenate(
        [e_row, z_src, s_src, jnp.zeros((1, 64 - V - 2), jnp.float32),
         pooled_src], axis=1)


def _sample_body(u_ref, logits_ref, gt_ref, rs_ref, r20_ref, sm_ref, w2_ref,
                 seq_ref, umh_ref, mut_ref, acc_ref, ms_ref, mh_ref):
    Bg = u_ref.shape[0]
    LV = u_ref.shape[1]
    L = seq_ref.shape[1]
    V = _NUM_TOKENS
    H = r20_ref.shape[1]
    inv_l = 1.0 / L
    u_c = jnp.maximum(u_ref[...], 1e-9)
    e = -jnp.log(u_c)
    y = logits_ref[...] - jnp.log(e)
    rowmax = jnp.max(y, axis=1, keepdims=True)
    idx = lax.broadcasted_iota(jnp.int32, (Bg, LV), 1)
    m = jnp.min(jnp.where(y == rowmax, idx, LV), axis=1, keepdims=True)
    res = m // V
    aa = m - res * V

    pos = lax.broadcasted_iota(jnp.int32, (Bg, L), 1)
    hit = pos == res
    mut_ref[...] = jnp.where(hit, aa, seq_ref[...])

    hitf = hit.astype(jnp.float32)
    feat = lax.dot_general(hitf, gt_ref[...], (((1,), (0,)), ((), ())),
                           preferred_element_type=jnp.float32,
                           precision=lax.Precision.HIGHEST)
    rsr = lax.dot_general(hitf, rs_ref[...], (((1,), (0,)), ((), ())),
                          preferred_element_type=jnp.float32,
                          precision=lax.Precision.HIGHEST)
    lane32 = lax.broadcasted_iota(jnp.int32, (Bg, 32), 1)
    lane20 = lane32[:, :V]
    rowvals = feat[:, :V]
    logit_sel = jnp.sum(jnp.where(lane20 == aa, rowvals, 0.0),
                        axis=1, keepdims=True)
    e_r = jnp.sum(jnp.where(lane32 == V, feat, 0.0), axis=1, keepdims=True)

    ohaa = (lane20 == aa).astype(jnp.float32)
    raa = lax.dot_general(ohaa, r20_ref[...], (((1,), (0,)), ((), ())),
                          preferred_element_type=jnp.float32,
                          precision=lax.Precision.HIGHEST)

    sm = sm_ref[...]
    lane128 = lax.broadcasted_iota(jnp.int32, (Bg, 128), 1)
    e_aa = jnp.sum(jnp.where(lane128 == aa, sm, 0.0), axis=1, keepdims=True)
    z_src = jnp.sum(jnp.where(lane128 == V, sm, 0.0), axis=1, keepdims=True)
    s_src = jnp.sum(jnp.where(lane128 == V + 1, sm, 0.0), axis=1, keepdims=True)
    pooled_src = sm[:, 64:64 + H]

    pooled = pooled_src + (raa - rsr) * inv_l
    w2b = w2_ref[...].astype(jnp.bfloat16).astype(jnp.float32)
    pb = pooled.astype(jnp.bfloat16).astype(jnp.float32)
    mut_score = jnp.sum(pb * w2b, axis=1, keepdims=True)

    delta_score = mut_score - s_src
    z_mut = z_src - e_r + e_aa
    accept = jnp.exp(delta_score) * z_src / (z_mut * jnp.exp(logit_sel))
    acc_ref[...] = accept
    ms_ref[...] = mut_score
    mh_ref[...] = (accept < umh_ref[...]).astype(jnp.float32)


@jax.jit
def kernel(seq_tokens, u_gumbel, u_mh, W1, w2):
    L = seq_tokens.shape[0]
    G = u_gumbel.shape[0]
    V = _NUM_TOKENS
    H = W1.shape[1]
    Bg = 256

    gtable, rseq, r20, smalls = pl.pallas_call(
        _tables_body,
        out_shape=[
            jax.ShapeDtypeStruct((L, 32), jnp.float32),
            jax.ShapeDtypeStruct((L, H), jnp.float32),
            jax.ShapeDtypeStruct((V, H), jnp.float32),
            jax.ShapeDtypeStruct((1, 128), jnp.float32),
        ],
    )(seq_tokens.reshape(L, 1), W1, w2.reshape(1, H))

    logits_flat = gtable[:, :V].reshape(1, L * V)

    mutants, accept, mscore, mhf = pl.pallas_call(
        _sample_body,
        grid=(G // Bg,),
        in_specs=[
            pl.BlockSpec((Bg, L * V), lambda i: (i, 0)),
            pl.BlockSpec((1, L * V), lambda i: (0, 0)),
            pl.BlockSpec((L, 32), lambda i: (0, 0)),
            pl.BlockSpec((L, H), lambda i: (0, 0)),
            pl.BlockSpec((V, H), lambda i: (0, 0)),
            pl.BlockSpec((1, 128), lambda i: (0, 0)),
            pl.BlockSpec((1, H), lambda i: (0, 0)),
            pl.BlockSpec((1, L), lambda i: (0, 0)),
            pl.BlockSpec((Bg, 1), lambda i: (i, 0)),
        ],
        out_specs=[
            pl.BlockSpec((Bg, L), lambda i: (i, 0)),
            pl.BlockSpec((Bg, 1), lambda i: (i, 0)),
            pl.BlockSpec((Bg, 1), lambda i: (i, 0)),
            pl.BlockSpec((Bg, 1), lambda i: (i, 0)),
        ],
        out_shape=[
            jax.ShapeDtypeStruct((G, L), seq_tokens.dtype),
            jax.ShapeDtypeStruct((G, 1), jnp.float32),
            jax.ShapeDtypeStruct((G, 1), jnp.float32),
            jax.ShapeDtypeStruct((G, 1), jnp.float32),
        ],
    )(u_gumbel, logits_flat, gtable, rseq, r20, smalls, w2.reshape(1, H),
      seq_tokens.reshape(1, L), u_mh.reshape(G, 1))

    return (accept.reshape(G), mhf.reshape(G).astype(bool),
            mutants, mscore.reshape(G))


# speedup vs baseline: 1.0798x; 1.0798x over previous
import jax
import jax.numpy as jnp
from jax import lax
from jax.experimental import pallas as pl

_NUM_TOKENS = 20
_TEMP = 0.1


def _tables_body(seq_ref, w1_ref, w2_ref, gt_ref, r20_ref, sm_ref):
    L = seq_ref.shape[0]
    V = _NUM_TOKENS
    W1 = w1_ref[...]
    w2 = w2_ref[...]
    inv_l = 1.0 / L
    w1b16 = W1.astype(jnp.bfloat16)
    W1b = w1b16.astype(jnp.float32)
    r20 = jnp.maximum(W1b, 0.0)

    M = jnp.where(W1b > 0.0, w2 * inv_l, 0.0)
    Mb = M.astype(jnp.bfloat16).astype(jnp.float32)
    DT = jnp.concatenate(
        [jnp.sum(W1b * Mb[t:t + 1, :], axis=1, keepdims=True)
         for t in range(V)], axis=1)
    eye = (lax.broadcasted_iota(jnp.int32, (V, V), 0)
           == lax.broadcasted_iota(jnp.int32, (V, V), 1))
    ddiag_row = jnp.sum(jnp.where(eye, DT, 0.0), axis=0, keepdims=True)
    e_row = jnp.sum(jnp.exp((DT - ddiag_row) / _TEMP), axis=0, keepdims=True)

    seq = seq_ref[...]
    ohf = (lax.broadcasted_iota(jnp.int32, (L, V), 1) == seq).astype(jnp.float32)
    rows = lax.dot_general(ohf, DT, (((1,), (1,)), ((), ())),
                           preferred_element_type=jnp.float32,
                           precision=lax.Precision.HIGHEST)
    dll = jnp.sum(rows * ohf, axis=1, keepdims=True)
    logits2d = (rows - dll) / _TEMP
    e_seq = jnp.sum(ohf * e_row, axis=1, keepdims=True)
    z_src = jnp.sum(e_seq, axis=0, keepdims=True)

    counts = jnp.sum(ohf, axis=0, keepdims=True)
    pooled_src = lax.dot_general(counts, r20, (((1,), (0,)), ((), ())),
                                 preferred_element_type=jnp.float32,
                                 precision=lax.Precision.HIGHEST) * inv_l
    w2b = w2.astype(jnp.bfloat16).astype(jnp.float32)
    psb = pooled_src.astype(jnp.bfloat16).astype(jnp.float32)
    s_src = jnp.sum(psb * w2b, axis=1, keepdims=True)

    gt_ref[...] = jnp.concatenate(
        [logits2d, e_seq, jnp.zeros((L, 11), jnp.float32)], axis=1)
    r20_ref[...] = r20
    sm_ref[...] = jnp.concatenate(
        [e_row, z_src, s_src, jnp.zeros((1, 64 - V - 2), jnp.float32),
         pooled_src], axis=1)


def _sample_body(u_ref, logits_ref, gt_ref, r20_ref, sm_ref, w2_ref,
                 seq_ref, umh_ref, mut_ref, acc_ref, ms_ref, mh_ref):
    Bg = u_ref.shape[0]
    LV = u_ref.shape[1]
    L = seq_ref.shape[1]
    V = _NUM_TOKENS
    H = r20_ref.shape[1]
    inv_l = 1.0 / L
    u_c = jnp.maximum(u_ref[...], 1e-9)
    e = -jnp.log(u_c)
    y = logits_ref[...] - jnp.log(e)
    rowmax = jnp.max(y, axis=1, keepdims=True)
    idx = lax.broadcasted_iota(jnp.int32, (Bg, LV), 1)
    m = jnp.min(jnp.where(y == rowmax, idx, LV), axis=1, keepdims=True)
    res = m // V
    aa = m - res * V

    pos = lax.broadcasted_iota(jnp.int32, (Bg, L), 1)
    hit = pos == res
    mut_ref[...] = jnp.where(hit, aa, seq_ref[...])

    hitf = hit.astype(jnp.float32)
    feat = lax.dot_general(hitf, gt_ref[...], (((1,), (0,)), ((), ())),
                           preferred_element_type=jnp.float32,
                           precision=lax.Precision.HIGHEST)
    s_r = jnp.sum(jnp.where(hit, seq_ref[...], 0), axis=1, keepdims=True)
    lane32 = lax.broadcasted_iota(jnp.int32, (Bg, 32), 1)
    lane20 = lane32[:, :V]
    rowvals = feat[:, :V]
    logit_sel = jnp.sum(jnp.where(lane20 == aa, rowvals, 0.0),
                        axis=1, keepdims=True)
    e_r = jnp.sum(jnp.where(lane32 == V, feat, 0.0), axis=1, keepdims=True)

    ohaa = (lane20 == aa).astype(jnp.float32)
    raa = lax.dot_general(ohaa, r20_ref[...], (((1,), (0,)), ((), ())),
                          preferred_element_type=jnp.float32,
                          precision=lax.Precision.HIGHEST)
    ohsr = (lane20 == s_r).astype(jnp.float32)
    rsr = lax.dot_general(ohsr, r20_ref[...], (((1,), (0,)), ((), ())),
                          preferred_element_type=jnp.float32,
                          precision=lax.Precision.HIGHEST)

    sm = sm_ref[...]
    lane128 = lax.broadcasted_iota(jnp.int32, (Bg, 128), 1)
    e_aa = jnp.sum(jnp.where(lane128 == aa, sm, 0.0), axis=1, keepdims=True)
    z_src = jnp.sum(jnp.where(lane128 == V, sm, 0.0), axis=1, keepdims=True)
    s_src = jnp.sum(jnp.where(lane128 == V + 1, sm, 0.0), axis=1, keepdims=True)
    pooled_src = sm[:, 64:64 + H]

    pooled = pooled_src + (raa - rsr) * inv_l
    w2b = w2_ref[...].astype(jnp.bfloat16).astype(jnp.float32)
    pb = pooled.astype(jnp.bfloat16).astype(jnp.float32)
    mut_score = jnp.sum(pb * w2b, axis=1, keepdims=True)

    delta_score = mut_score - s_src
    z_mut = z_src - e_r + e_aa
    accept = jnp.exp(delta_score) * z_src / (z_mut * jnp.exp(logit_sel))
    acc_ref[...] = accept
    ms_ref[...] = mut_score
    mh_ref[...] = (accept < umh_ref[...]).astype(jnp.float32)


@jax.jit
def kernel(seq_tokens, u_gumbel, u_mh, W1, w2):
    L = seq_tokens.shape[0]
    G = u_gumbel.shape[0]
    V = _NUM_TOKENS
    H = W1.shape[1]
    Bg = 256

    gtable, r20, smalls = pl.pallas_call(
        _tables_body,
        out_shape=[
            jax.ShapeDtypeStruct((L, 32), jnp.float32),
            jax.ShapeDtypeStruct((V, H), jnp.float32),
            jax.ShapeDtypeStruct((1, 128), jnp.float32),
        ],
    )(seq_tokens.reshape(L, 1), W1, w2.reshape(1, H))

    logits_flat = gtable[:, :V].reshape(1, L * V)

    mutants, accept, mscore, mhf = pl.pallas_call(
        _sample_body,
        grid=(G // Bg,),
        in_specs=[
            pl.BlockSpec((Bg, L * V), lambda i: (i, 0)),
            pl.BlockSpec((1, L * V), lambda i: (0, 0)),
            pl.BlockSpec((L, 32), lambda i: (0, 0)),
            pl.BlockSpec((V, H), lambda i: (0, 0)),
            pl.BlockSpec((1, 128), lambda i: (0, 0)),
            pl.BlockSpec((1, H), lambda i: (0, 0)),
            pl.BlockSpec((1, L), lambda i: (0, 0)),
            pl.BlockSpec((Bg, 1), lambda i: (i, 0)),
        ],
        out_specs=[
            pl.BlockSpec((Bg, L), lambda i: (i, 0)),
            pl.BlockSpec((Bg, 1), lambda i: (i, 0)),
            pl.BlockSpec((Bg, 1), lambda i: (i, 0)),
            pl.BlockSpec((Bg, 1), lambda i: (i, 0)),
        ],
        out_shape=[
            jax.ShapeDtypeStruct((G, L), seq_tokens.dtype),
            jax.ShapeDtypeStruct((G, 1), jnp.float32),
            jax.ShapeDtypeStruct((G, 1), jnp.float32),
            jax.ShapeDtypeStruct((G, 1), jnp.float32),
        ],
    )(u_gumbel, logits_flat, gtable, r20, smalls, w2.reshape(1, H),
      seq_tokens.reshape(1, L), u_mh.reshape(G, 1))

    return (accept.reshape(G), mhf.reshape(G).astype(bool),
            mutants, mscore.reshape(G))
